# baseline (speedup 1.0000x reference)
import functools

import jax
import jax.numpy as jnp
from jax import lax
from jax.experimental import pallas as pl
from jax.experimental.pallas import tpu as pltpu

POOL_N = 64
TOPK_N = 8
LEN_N = 5
D_N = 768
H_N = 4096
B_N = 128
S_N = 197

NT = 8
TH = H_N // NT


def _dense_body(cls_ref, pk_ref, eps_ref,
                w1_ref, b1_ref, wm_ref, bm_ref, wv_ref, bv_ref,
                d1_ref, db1_ref, d2_ref, db2_ref,
                sim_ref, mean_ref, lv_ref, syn_ref, idx_ref, rs_ref,
                acc_mean, acc_lv, z_ref, acc_syn):
    i = pl.program_id(0)

    @pl.when(i == 0)
    def _init():
        acc_mean[...] = jnp.zeros_like(acc_mean)
        acc_lv[...] = jnp.zeros_like(acc_lv)
        acc_syn[...] = jnp.zeros_like(acc_syn)
        x = cls_ref[...]
        xn = x * lax.rsqrt(jnp.maximum(
            jnp.sum(x * x, axis=1, keepdims=True), 1e-12))
        p = pk_ref[...]
        pn = p * lax.rsqrt(jnp.maximum(
            jnp.sum(p * p, axis=1, keepdims=True), 1e-12))
        sim = lax.dot_general(xn, pn, (((1,), (1,)), ((), ())),
                              preferred_element_type=jnp.float32)
        sim_ref[...] = sim
        iot = lax.broadcasted_iota(jnp.int32, (B_N, POOL_N), 1)
        work = sim
        total = jnp.float32(0.0)
        cols = []
        for _ in range(TOPK_N):
            m = jnp.max(work, axis=1, keepdims=True)
            total = total + jnp.sum(m)
            cand = jnp.where(work == m, iot, POOL_N)
            aidx = jnp.min(cand, axis=1)
            cols.append(aidx.reshape(B_N, 1))
            work = jnp.where(iot == aidx[:, None], -jnp.inf, work)
        idx_ref[...] = jnp.concatenate(cols, axis=1)
        rs_ref[...] = jnp.reshape(total / jnp.float32(B_N), (1, 1))

    @pl.when(i < NT)
    def _encoder():
        h = jnp.maximum(
            lax.dot_general(cls_ref[...], w1_ref[...],
                            (((1,), (1,)), ((), ())),
                            preferred_element_type=jnp.float32)
            + b1_ref[...][None, :], 0.0)
        acc_mean[...] += lax.dot_general(h, wm_ref[...],
                                         (((1,), (1,)), ((), ())),
                                         preferred_element_type=jnp.float32)
        acc_lv[...] += lax.dot_general(h, wv_ref[...],
                                       (((1,), (1,)), ((), ())),
                                       preferred_element_type=jnp.float32)

    @pl.when(i == NT - 1)
    def _reparam():
        mean = acc_mean[...] + bm_ref[...][None, :]
        log_var = acc_lv[...] + bv_ref[...][None, :]
        mean_ref[...] = mean
        lv_ref[...] = log_var
        z_ref[...] = mean + jnp.exp(0.5 * log_var) * eps_ref[...]

    @pl.when(i >= NT)
    def _decoder():
        hd = jnp.maximum(
            lax.dot_general(z_ref[...], d1_ref[...],
                            (((1,), (1,)), ((), ())),
                            preferred_element_type=jnp.float32)
            + db1_ref[...][None, :], 0.0)
        acc_syn[...] += lax.dot_general(hd, d2_ref[...],
                                        (((1,), (1,)), ((), ())),
                                        preferred_element_type=jnp.float32)

    @pl.when(i == 2 * NT - 1)
    def _finish():
        syn_ref[...] = acc_syn[...] + db2_ref[...][None, :]


def _assemble_body(idx_sref, prompt_ref, x_ref, syn_ref, cls_ref, out_ref):
    b = pl.program_id(0)
    out_ref[0, 41:, :] = x_ref[0]
    out_ref[0, 40:41, :] = cls_ref[0]
    s = syn_ref[0]
    for k in range(TOPK_N):
        pidx = idx_sref[b, k]
        out_ref[0, k * LEN_N:(k + 1) * LEN_N, :] = prompt_ref[pidx] + s


def kernel(is_training, x_embed, cls_features, prompt, prompt_key, frequency,
           W1, b1, Wm, bm, Wv, bv, D1, db1, D2, db2, epsilon):
    del is_training, frequency

    enc_t = lambda i: jnp.where(i < NT, i, 0)
    dec_t = lambda i: jnp.where(i >= NT, i - NT, 0)

    sim, mean, log_var, synth, idx, rs = pl.pallas_call(
        _dense_body,
        grid=(2 * NT,),
        in_specs=[
            pl.BlockSpec((B_N, D_N), lambda i: (0, 0)),
            pl.BlockSpec((POOL_N, D_N), lambda i: (0, 0)),
            pl.BlockSpec((B_N, D_N), lambda i: (0, 0)),
            pl.BlockSpec((TH, D_N), lambda i: (enc_t(i), 0)),
            pl.BlockSpec((TH,), lambda i: (enc_t(i),)),
            pl.BlockSpec((D_N, TH), lambda i: (0, enc_t(i))),
            pl.BlockSpec((D_N,), lambda i: (0,)),
            pl.BlockSpec((D_N, TH), lambda i: (0, enc_t(i))),
            pl.BlockSpec((D_N,), lambda i: (0,)),
            pl.BlockSpec((TH, D_N), lambda i: (dec_t(i), 0)),
            pl.BlockSpec((TH,), lambda i: (dec_t(i),)),
            pl.BlockSpec((D_N, TH), lambda i: (0, dec_t(i))),
            pl.BlockSpec((D_N,), lambda i: (0,)),
        ],
        out_specs=[
            pl.BlockSpec((B_N, POOL_N), lambda i: (0, 0)),
            pl.BlockSpec((B_N, D_N), lambda i: (0, 0)),
            pl.BlockSpec((B_N, D_N), lambda i: (0, 0)),
            pl.BlockSpec((B_N, D_N), lambda i: (0, 0)),
            pl.BlockSpec((B_N, TOPK_N), lambda i: (0, 0)),
            pl.BlockSpec((1, 1), lambda i: (0, 0)),
        ],
        out_shape=[
            jax.ShapeDtypeStruct((B_N, POOL_N), jnp.float32),
            jax.ShapeDtypeStruct((B_N, D_N), jnp.float32),
            jax.ShapeDtypeStruct((B_N, D_N), jnp.float32),
            jax.ShapeDtypeStruct((B_N, D_N), jnp.float32),
            jax.ShapeDtypeStruct((B_N, TOPK_N), jnp.int32),
            jax.ShapeDtypeStruct((1, 1), jnp.float32),
        ],
        scratch_shapes=[
            pltpu.VMEM((B_N, D_N), jnp.float32),
            pltpu.VMEM((B_N, D_N), jnp.float32),
            pltpu.VMEM((B_N, D_N), jnp.float32),
            pltpu.VMEM((B_N, D_N), jnp.float32),
        ],
    )(cls_features, prompt_key, epsilon,
      W1, b1, Wm, bm, Wv, bv, D1, db1, D2, db2)

    T_OUT = 1 + TOPK_N * LEN_N + S_N

    prompted = pl.pallas_call(
        _assemble_body,
        grid_spec=pltpu.PrefetchScalarGridSpec(
            num_scalar_prefetch=1,
            grid=(B_N,),
            in_specs=[
                pl.BlockSpec((POOL_N, LEN_N, D_N), lambda b, idx: (0, 0, 0)),
                pl.BlockSpec((1, S_N, D_N), lambda b, idx: (b, 0, 0)),
                pl.BlockSpec((1, 1, D_N), lambda b, idx: (b, 0, 0)),
                pl.BlockSpec((1, 1, D_N), lambda b, idx: (b, 0, 0)),
            ],
            out_specs=pl.BlockSpec((1, T_OUT, D_N), lambda b, idx: (b, 0, 0)),
        ),
        out_shape=jax.ShapeDtypeStruct((B_N, T_OUT, D_N), jnp.float32),
    )(idx, prompt, x_embed,
      synth.reshape(B_N, 1, D_N), cls_features.reshape(B_N, 1, D_N))

    return (prompted, rs.reshape(()), sim, synth, mean, log_var, idx)


# speedup vs baseline: 1.1933x; 1.1933x over previous
import functools

import jax
import jax.numpy as jnp
from jax import lax
from jax.experimental import pallas as pl
from jax.experimental.pallas import tpu as pltpu

POOL_N = 64
TOPK_N = 8
LEN_N = 5
D_N = 768
H_N = 4096
B_N = 128
S_N = 197

NT = 8
TH = H_N // NT


def _dense_body(cls_ref, pk_ref, eps_ref,
                w1_ref, b1_ref, wm_ref, bm_ref, wv_ref, bv_ref,
                d1_ref, db1_ref, d2_ref, db2_ref,
                sim_ref, mean_ref, lv_ref, syn_ref, idx_ref, rs_ref,
                acc_mean, acc_lv, z_ref, acc_syn):
    i = pl.program_id(0)

    @pl.when(i == 0)
    def _init():
        acc_mean[...] = jnp.zeros_like(acc_mean)
        acc_lv[...] = jnp.zeros_like(acc_lv)
        acc_syn[...] = jnp.zeros_like(acc_syn)
        x = cls_ref[...]
        xn = x * lax.rsqrt(jnp.maximum(
            jnp.sum(x * x, axis=1, keepdims=True), 1e-12))
        p = pk_ref[...]
        pn = p * lax.rsqrt(jnp.maximum(
            jnp.sum(p * p, axis=1, keepdims=True), 1e-12))
        sim = lax.dot_general(xn, pn, (((1,), (1,)), ((), ())),
                              preferred_element_type=jnp.float32)
        sim_ref[...] = sim
        iot = lax.broadcasted_iota(jnp.int32, (B_N, POOL_N), 1)
        work = sim
        total = jnp.float32(0.0)
        cols = []
        for _ in range(TOPK_N):
            m = jnp.max(work, axis=1, keepdims=True)
            total = total + jnp.sum(m)
            cand = jnp.where(work == m, iot, POOL_N)
            aidx = jnp.min(cand, axis=1)
            cols.append(aidx.reshape(B_N, 1))
            work = jnp.where(iot == aidx[:, None], -jnp.inf, work)
        idx_ref[...] = jnp.concatenate(cols, axis=1)
        rs_ref[...] = jnp.reshape(total / jnp.float32(B_N), (1, 1))

    @pl.when(i < NT)
    def _encoder():
        h = jnp.maximum(
            lax.dot_general(cls_ref[...], w1_ref[...],
                            (((1,), (1,)), ((), ())),
                            preferred_element_type=jnp.float32)
            + b1_ref[...][None, :], 0.0)
        acc_mean[...] += lax.dot_general(h, wm_ref[...],
                                         (((1,), (1,)), ((), ())),
                                         preferred_element_type=jnp.float32)
        acc_lv[...] += lax.dot_general(h, wv_ref[...],
                                       (((1,), (1,)), ((), ())),
                                       preferred_element_type=jnp.float32)

    @pl.when(i == NT - 1)
    def _reparam():
        mean = acc_mean[...] + bm_ref[...][None, :]
        log_var = acc_lv[...] + bv_ref[...][None, :]
        mean_ref[...] = mean
        lv_ref[...] = log_var
        z_ref[...] = mean + jnp.exp(0.5 * log_var) * eps_ref[...]

    @pl.when(i >= NT)
    def _decoder():
        hd = jnp.maximum(
            lax.dot_general(z_ref[...], d1_ref[...],
                            (((1,), (1,)), ((), ())),
                            preferred_element_type=jnp.float32)
            + db1_ref[...][None, :], 0.0)
        acc_syn[...] += lax.dot_general(hd, d2_ref[...],
                                        (((1,), (1,)), ((), ())),
                                        preferred_element_type=jnp.float32)

    @pl.when(i == 2 * NT - 1)
    def _finish():
        syn_ref[...] = acc_syn[...] + db2_ref[...][None, :]


BB = 8


def _assemble_body(idx_sref, prompt_ref, x_ref, syn_ref, cls_ref, out_ref):
    g = pl.program_id(0)
    for bb in range(BB):
        out_ref[bb, 41:, :] = x_ref[bb]
        out_ref[bb, 40:41, :] = cls_ref[bb]
        s = syn_ref[bb]
        for k in range(TOPK_N):
            pidx = idx_sref[g * BB + bb, k]
            out_ref[bb, k * LEN_N:(k + 1) * LEN_N, :] = prompt_ref[pidx] + s


def kernel(is_training, x_embed, cls_features, prompt, prompt_key, frequency,
           W1, b1, Wm, bm, Wv, bv, D1, db1, D2, db2, epsilon):
    del is_training, frequency

    enc_t = lambda i: jnp.where(i < NT, i, 0)
    dec_t = lambda i: jnp.where(i >= NT, i - NT, 0)

    sim, mean, log_var, synth, idx, rs = pl.pallas_call(
        _dense_body,
        grid=(2 * NT,),
        in_specs=[
            pl.BlockSpec((B_N, D_N), lambda i: (0, 0)),
            pl.BlockSpec((POOL_N, D_N), lambda i: (0, 0)),
            pl.BlockSpec((B_N, D_N), lambda i: (0, 0)),
            pl.BlockSpec((TH, D_N), lambda i: (enc_t(i), 0)),
            pl.BlockSpec((TH,), lambda i: (enc_t(i),)),
            pl.BlockSpec((D_N, TH), lambda i: (0, enc_t(i))),
            pl.BlockSpec((D_N,), lambda i: (0,)),
            pl.BlockSpec((D_N, TH), lambda i: (0, enc_t(i))),
            pl.BlockSpec((D_N,), lambda i: (0,)),
            pl.BlockSpec((TH, D_N), lambda i: (dec_t(i), 0)),
            pl.BlockSpec((TH,), lambda i: (dec_t(i),)),
            pl.BlockSpec((D_N, TH), lambda i: (0, dec_t(i))),
            pl.BlockSpec((D_N,), lambda i: (0,)),
        ],
        out_specs=[
            pl.BlockSpec((B_N, POOL_N), lambda i: (0, 0)),
            pl.BlockSpec((B_N, D_N), lambda i: (0, 0)),
            pl.BlockSpec((B_N, D_N), lambda i: (0, 0)),
            pl.BlockSpec((B_N, D_N), lambda i: (0, 0)),
            pl.BlockSpec((B_N, TOPK_N), lambda i: (0, 0)),
            pl.BlockSpec((1, 1), lambda i: (0, 0)),
        ],
        out_shape=[
            jax.ShapeDtypeStruct((B_N, POOL_N), jnp.float32),
            jax.ShapeDtypeStruct((B_N, D_N), jnp.float32),
            jax.ShapeDtypeStruct((B_N, D_N), jnp.float32),
            jax.ShapeDtypeStruct((B_N, D_N), jnp.float32),
            jax.ShapeDtypeStruct((B_N, TOPK_N), jnp.int32),
            jax.ShapeDtypeStruct((1, 1), jnp.float32),
        ],
        scratch_shapes=[
            pltpu.VMEM((B_N, D_N), jnp.float32),
            pltpu.VMEM((B_N, D_N), jnp.float32),
            pltpu.VMEM((B_N, D_N), jnp.float32),
            pltpu.VMEM((B_N, D_N), jnp.float32),
        ],
    )(cls_features, prompt_key, epsilon,
      W1, b1, Wm, bm, Wv, bv, D1, db1, D2, db2)

    T_OUT = 1 + TOPK_N * LEN_N + S_N

    prompted = pl.pallas_call(
        _assemble_body,
        grid_spec=pltpu.PrefetchScalarGridSpec(
            num_scalar_prefetch=1,
            grid=(B_N // BB,),
            in_specs=[
                pl.BlockSpec((POOL_N, LEN_N, D_N), lambda b, idx: (0, 0, 0)),
                pl.BlockSpec((BB, S_N, D_N), lambda b, idx: (b, 0, 0)),
                pl.BlockSpec((BB, 1, D_N), lambda b, idx: (b, 0, 0)),
                pl.BlockSpec((BB, 1, D_N), lambda b, idx: (b, 0, 0)),
            ],
            out_specs=pl.BlockSpec((BB, T_OUT, D_N), lambda b, idx: (b, 0, 0)),
        ),
        out_shape=jax.ShapeDtypeStruct((B_N, T_OUT, D_N), jnp.float32),
    )(idx, prompt, x_embed,
      synth.reshape(B_N, 1, D_N), cls_features.reshape(B_N, 1, D_N))

    return (prompted, rs.reshape(()), sim, synth, mean, log_var, idx)


# speedup vs baseline: 1.2023x; 1.0075x over previous
import functools

import jax
import jax.numpy as jnp
from jax import lax
from jax.experimental import pallas as pl
from jax.experimental.pallas import tpu as pltpu

POOL_N = 64
TOPK_N = 8
LEN_N = 5
D_N = 768
H_N = 4096
B_N = 128
S_N = 197
T_OUT = 1 + TOPK_N * LEN_N + S_N

NT = 8
TH = H_N // NT

BB = 8
NG = B_N // BB


def _dense_body(cls_ref, pk_ref, eps_ref,
                w1_ref, b1_ref, wm_ref, bm_ref, wv_ref, bv_ref,
                d1_ref, db1_ref, d2_ref, db2_ref,
                sim_ref, mean_ref, lv_ref, syn_ref, idx_ref, rs_ref,
                acc_mean, acc_lv, z_ref, acc_syn):
    i = pl.program_id(0)

    @pl.when(i == 0)
    def _init():
        acc_mean[...] = jnp.zeros_like(acc_mean)
        acc_lv[...] = jnp.zeros_like(acc_lv)
        acc_syn[...] = jnp.zeros_like(acc_syn)
        x = cls_ref[...]
        xn = x * lax.rsqrt(jnp.maximum(
            jnp.sum(x * x, axis=1, keepdims=True), 1e-12))
        p = pk_ref[...]
        pn = p * lax.rsqrt(jnp.maximum(
            jnp.sum(p * p, axis=1, keepdims=True), 1e-12))
        sim = lax.dot_general(xn, pn, (((1,), (1,)), ((), ())),
                              preferred_element_type=jnp.float32)
        sim_ref[...] = sim
        iot = lax.broadcasted_iota(jnp.int32, (B_N, POOL_N), 1)
        work = sim
        total = jnp.float32(0.0)
        cols = []
        for _ in range(TOPK_N):
            m = jnp.max(work, axis=1, keepdims=True)
            total = total + jnp.sum(m)
            cand = jnp.where(work == m, iot, POOL_N)
            aidx = jnp.min(cand, axis=1)
            cols.append(aidx.reshape(B_N, 1))
            work = jnp.where(iot == aidx[:, None], -jnp.inf, work)
        idx_ref[...] = jnp.concatenate(cols, axis=1)
        rs_ref[...] = jnp.reshape(total / jnp.float32(B_N), (1, 1))

    @pl.when(i < NT)
    def _encoder():
        h = jnp.maximum(
            lax.dot_general(cls_ref[...], w1_ref[...],
                            (((1,), (1,)), ((), ())),
                            preferred_element_type=jnp.float32)
            + b1_ref[...][None, :], 0.0)
        acc_mean[...] += lax.dot_general(h, wm_ref[...],
                                         (((1,), (1,)), ((), ())),
                                         preferred_element_type=jnp.float32)
        acc_lv[...] += lax.dot_general(h, wv_ref[...],
                                       (((1,), (1,)), ((), ())),
                                       preferred_element_type=jnp.float32)

    @pl.when(i == NT - 1)
    def _reparam():
        mean = acc_mean[...] + bm_ref[...][None, :]
        log_var = acc_lv[...] + bv_ref[...][None, :]
        mean_ref[...] = mean
        lv_ref[...] = log_var
        z_ref[...] = mean + jnp.exp(0.5 * log_var) * eps_ref[...]

    @pl.when(i >= NT)
    def _decoder():
        hd = jnp.maximum(
            lax.dot_general(z_ref[...], d1_ref[...],
                            (((1,), (1,)), ((), ())),
                            preferred_element_type=jnp.float32)
            + db1_ref[...][None, :], 0.0)
        acc_syn[...] += lax.dot_general(hd, d2_ref[...],
                                        (((1,), (1,)), ((), ())),
                                        preferred_element_type=jnp.float32)

    @pl.when(i == 2 * NT - 1)
    def _finish():
        syn_ref[...] = acc_syn[...] + db2_ref[...][None, :]


def _assemble_body(idx_sref, prompt_ref, x_ref, syn_ref, cls_ref, out_ref,
                   obuf, sem):
    g = pl.program_id(0)
    buf = lax.rem(g, 2)

    @pl.when(g >= 2)
    def _drain():
        pltpu.make_async_copy(
            obuf.at[buf], out_ref.at[pl.ds((g - 2) * BB, BB)],
            sem.at[buf]).wait()

    obuf[buf, :, 41:, :] = x_ref[...]
    obuf[buf, :, 40:41, :] = cls_ref[...]
    for bb in range(BB):
        s = syn_ref[bb, 0, :]
        for k in range(TOPK_N):
            pidx = idx_sref[g * BB + bb, k]
            obuf[buf, bb, k * LEN_N:(k + 1) * LEN_N, :] = (
                prompt_ref[pidx] + s[None, :])

    pltpu.make_async_copy(
        obuf.at[buf], out_ref.at[pl.ds(g * BB, BB)], sem.at[buf]).start()

    @pl.when(g == NG - 1)
    def _final_drain():
        pltpu.make_async_copy(
            obuf.at[1 - buf], out_ref.at[pl.ds((g - 1) * BB, BB)],
            sem.at[1 - buf]).wait()
        pltpu.make_async_copy(
            obuf.at[buf], out_ref.at[pl.ds(g * BB, BB)],
            sem.at[buf]).wait()


def kernel(is_training, x_embed, cls_features, prompt, prompt_key, frequency,
           W1, b1, Wm, bm, Wv, bv, D1, db1, D2, db2, epsilon):
    del is_training, frequency

    enc_t = lambda i: jnp.where(i < NT, i, 0)
    dec_t = lambda i: jnp.where(i >= NT, i - NT, 0)

    sim, mean, log_var, synth, idx, rs = pl.pallas_call(
        _dense_body,
        grid=(2 * NT,),
        in_specs=[
            pl.BlockSpec((B_N, D_N), lambda i: (0, 0)),
            pl.BlockSpec((POOL_N, D_N), lambda i: (0, 0)),
            pl.BlockSpec((B_N, D_N), lambda i: (0, 0)),
            pl.BlockSpec((TH, D_N), lambda i: (enc_t(i), 0)),
            pl.BlockSpec((TH,), lambda i: (enc_t(i),)),
            pl.BlockSpec((D_N, TH), lambda i: (0, enc_t(i))),
            pl.BlockSpec((D_N,), lambda i: (0,)),
            pl.BlockSpec((D_N, TH), lambda i: (0, enc_t(i))),
            pl.BlockSpec((D_N,), lambda i: (0,)),
            pl.BlockSpec((TH, D_N), lambda i: (dec_t(i), 0)),
            pl.BlockSpec((TH,), lambda i: (dec_t(i),)),
            pl.BlockSpec((D_N, TH), lambda i: (0, dec_t(i))),
            pl.BlockSpec((D_N,), lambda i: (0,)),
        ],
        out_specs=[
            pl.BlockSpec((B_N, POOL_N), lambda i: (0, 0)),
            pl.BlockSpec((B_N, D_N), lambda i: (0, 0)),
            pl.BlockSpec((B_N, D_N), lambda i: (0, 0)),
            pl.BlockSpec((B_N, D_N), lambda i: (0, 0)),
            pl.BlockSpec((B_N, TOPK_N), lambda i: (0, 0)),
            pl.BlockSpec((1, 1), lambda i: (0, 0)),
        ],
        out_shape=[
            jax.ShapeDtypeStruct((B_N, POOL_N), jnp.float32),
            jax.ShapeDtypeStruct((B_N, D_N), jnp.float32),
            jax.ShapeDtypeStruct((B_N, D_N), jnp.float32),
            jax.ShapeDtypeStruct((B_N, D_N), jnp.float32),
            jax.ShapeDtypeStruct((B_N, TOPK_N), jnp.int32),
            jax.ShapeDtypeStruct((1, 1), jnp.float32),
        ],
        scratch_shapes=[
            pltpu.VMEM((B_N, D_N), jnp.float32),
            pltpu.VMEM((B_N, D_N), jnp.float32),
            pltpu.VMEM((B_N, D_N), jnp.float32),
            pltpu.VMEM((B_N, D_N), jnp.float32),
        ],
    )(cls_features, prompt_key, epsilon,
      W1, b1, Wm, bm, Wv, bv, D1, db1, D2, db2)

    prompted = pl.pallas_call(
        _assemble_body,
        grid_spec=pltpu.PrefetchScalarGridSpec(
            num_scalar_prefetch=1,
            grid=(NG,),
            in_specs=[
                pl.BlockSpec((POOL_N, LEN_N, D_N), lambda g, idx: (0, 0, 0)),
                pl.BlockSpec((BB, S_N, D_N), lambda g, idx: (g, 0, 0)),
                pl.BlockSpec((BB, 1, D_N), lambda g, idx: (g, 0, 0)),
                pl.BlockSpec((BB, 1, D_N), lambda g, idx: (g, 0, 0)),
            ],
            out_specs=pl.BlockSpec(memory_space=pl.ANY),
            scratch_shapes=[
                pltpu.VMEM((2, BB, T_OUT, D_N), jnp.float32),
                pltpu.SemaphoreType.DMA((2,)),
            ],
        ),
        out_shape=jax.ShapeDtypeStruct((B_N, T_OUT, D_N), jnp.float32),
    )(idx, prompt, x_embed,
      synth.reshape(B_N, 1, D_N), cls_features.reshape(B_N, 1, D_N))

    return (prompted, rs.reshape(()), sim, synth, mean, log_var, idx)
